# baseline (speedup 1.0000x reference)
import jax
import jax.numpy as jnp
from jax import lax
from jax.experimental import pallas as pl
from jax.experimental.pallas import tpu as pltpu
from jax.experimental.pallas import tpu_sc as plsc

B = 1024
N = 100000
D = 64
TEMP = 0.05
MOM = 0.2

NB = 5120
GRID = (N + NB - 1) // NB

NUM_SUBCORES = 16
LANES = 16
VPR = D // LANES
CHUNK = 512
NFULL = N // CHUNK
TAIL_BASE = NFULL * CHUNK
TAILBLK = 256


def _prep_body(xt_ref, tcol_ref, trow_ref, xnt_ref, xnst_ref,
               keepr_ref, keepc_ref):
    xt = xt_ref[...]
    ss = jnp.sum(xt * xt, axis=0, keepdims=True)
    xnt = xt * lax.rsqrt(jnp.maximum(ss, 1e-24))
    xnt_ref[...] = xnt
    xnst_ref[...] = xnt * (1.0 / TEMP)
    trow = trow_ref[...]
    win = jnp.full((1, B), -1, jnp.int32)
    for c in range(B // 128):
        tcol = tcol_ref[pl.ds(c * 128, 128), :]
        eq = tcol == trow
        iidx = lax.broadcasted_iota(jnp.int32, (128, B), 0) + (c * 128)
        win = jnp.maximum(
            win, jnp.max(jnp.where(eq, iidx, -1), axis=0, keepdims=True))
        jidx = lax.broadcasted_iota(jnp.int32, (128, B), 1)
        winc = jnp.max(jnp.where(eq, jidx, -1), axis=1, keepdims=True)
        ii = lax.broadcasted_iota(jnp.int32, (128, 1), 0) + (c * 128)
        keepc_ref[pl.ds(c * 128, 128), :] = (winc == ii).astype(jnp.int32)
    bidx = lax.broadcasted_iota(jnp.int32, (1, B), 1)
    keepr_ref[...] = (win == bidx).astype(jnp.int32)


_prep_call = pl.pallas_call(
    _prep_body,
    out_shape=[
        jax.ShapeDtypeStruct((D, B), jnp.float32),
        jax.ShapeDtypeStruct((D, B), jnp.float32),
        jax.ShapeDtypeStruct((1, B), jnp.int32),
        jax.ShapeDtypeStruct((B, 1), jnp.int32),
    ],
)


def _mm_body(xnst_ref, ft_ref, out_ref):
    out_ref[...] = lax.dot_general(
        ft_ref[...], xnst_ref[...], (((0,), (0,)), ((), ())),
        preferred_element_type=jnp.float32)


_mm_call = pl.pallas_call(
    _mm_body,
    grid=(GRID,),
    in_specs=[
        pl.BlockSpec((D, B), lambda j: (0, 0)),
        pl.BlockSpec((D, NB), lambda j: (0, j)),
    ],
    out_specs=[
        pl.BlockSpec((NB, B), lambda j: (j, 0)),
    ],
    out_shape=[
        jax.ShapeDtypeStruct((N, B), jnp.float32),
    ],
    compiler_params=pltpu.CompilerParams(
        dimension_semantics=("arbitrary",),
        vmem_limit_bytes=100 * 1024 * 1024,
    ),
)


def _rsqrt_newton(t16):
    i = lax.bitcast_convert_type(t16, jnp.int32)
    y = lax.bitcast_convert_type(jnp.int32(0x5F3759DF) - (i >> 1), jnp.float32)
    for _ in range(3):
        y = y * (1.5 - 0.5 * t16 * y * y)
    return y


def _sc_body(ft_hbm, xnt_hbm, tgt_hbm, keep_hbm, nft_hbm,
             xnt_v, tgt_v, keep_v, chunk_v, hitb_v):
    wid = lax.axis_index("s")
    pltpu.sync_copy(xnt_hbm, xnt_v)
    pltpu.sync_copy(tgt_hbm, tgt_v)
    pltpu.sync_copy(keep_hbm, keep_v)
    row16 = lax.iota(jnp.int32, 16)

    def process_chunk(cbase, width):
        cv = chunk_v.at[:, pl.ds(0, width)]
        pltpu.sync_copy(ft_hbm.at[:, pl.ds(cbase, width)], cv)
        cnt = jnp.int32(0)
        for q in range(B // LANES):
            tg = tgt_v[pl.ds(q * LANES, LANES)]
            kp = keep_v[pl.ds(q * LANES, LANES)]
            m = (tg >= cbase) & (tg < cbase + width) & (kp > 0)
            bv = row16 + (q * LANES)
            plsc.store_compressed(hitb_v.at[pl.ds(cnt, LANES)], bv, mask=m)
            cnt = cnt + plsc.all_reduce_population_count(m)[0]

        def fix_body(i, carry2):
            off = i * LANES
            valid = (row16 + off) < cnt
            b16 = jnp.where(valid, hitb_v[pl.ds(off, LANES)], 0)
            t16 = plsc.load_gather(tgt_v, [b16])
            c16 = t16 - cbase
            for lane in range(LANES):
                @pl.when(off + lane < cnt)
                def _():
                    b_s = b16[lane]
                    c_s = c16[lane]
                    cf = jnp.full((LANES,), c_s, jnp.int32)
                    bf = jnp.full((LANES,), b_s, jnp.int32)
                    us = []
                    ssum = None
                    for q4 in range(VPR):
                        ridx = row16 + (q4 * LANES)
                        g = plsc.load_gather(chunk_v, [ridx, cf])
                        xw = plsc.load_gather(xnt_v, [ridx, bf])
                        u = MOM * g + (1.0 - MOM) * xw
                        us.append(u)
                        p = u * u
                        ssum = p if ssum is None else ssum + p
                    tot = jnp.sum(ssum)
                    y = _rsqrt_newton(jnp.full((LANES,), tot, jnp.float32))
                    for q4 in range(VPR):
                        ridx = row16 + (q4 * LANES)
                        plsc.store_scatter(chunk_v, [ridx, cf], us[q4] * y)
            return carry2

        lax.fori_loop(0, (cnt + LANES - 1) // LANES, fix_body, 0)
        pltpu.sync_copy(cv, nft_hbm.at[:, pl.ds(cbase, width)])

    nmine = jnp.where(wid < (NFULL % NUM_SUBCORES),
                      NFULL // NUM_SUBCORES + 1, NFULL // NUM_SUBCORES)

    def chunk_body(ci, carry):
        cbase = (wid + ci * NUM_SUBCORES) * CHUNK
        process_chunk(cbase, CHUNK)
        return carry

    lax.fori_loop(0, nmine, chunk_body, 0)


_sc_update = pl.kernel(
    _sc_body,
    out_type=jax.ShapeDtypeStruct((D, N), jnp.float32),
    mesh=plsc.VectorSubcoreMesh(
        core_axis_name="c", subcore_axis_name="s",
        num_cores=1, num_subcores=NUM_SUBCORES),
    compiler_params=pltpu.CompilerParams(needs_layout_passes=False),
    scratch_types=[
        pltpu.VMEM((D, B), jnp.float32),
        pltpu.VMEM((B,), jnp.int32),
        pltpu.VMEM((B,), jnp.int32),
        pltpu.VMEM((D, CHUNK), jnp.float32),
        pltpu.VMEM((B + LANES,), jnp.int32),
    ],
)


def _tail_body(nft_ref, ftb_ref, tcol_ref, keepc_ref, xnt_ref, out_ref):
    del nft_ref
    ftb = ftb_ref[...]
    tcol = tcol_ref[...]
    keepc = keepc_ref[...]
    cidx = lax.broadcasted_iota(jnp.int32, (B, TAILBLK), 1) + TAIL_BASE
    h = ((tcol == cidx) & (keepc > 0)).astype(jnp.float32)
    g = lax.dot_general(ftb, h, (((1,), (1,)), ((), ())),
                        preferred_element_type=jnp.float32)
    upd = MOM * g + (1.0 - MOM) * xnt_ref[...]
    ss = jnp.sum(upd * upd, axis=0, keepdims=True)
    upd = upd * lax.rsqrt(jnp.maximum(ss, 1e-24))
    scat = lax.dot_general(upd, h, (((1,), (0,)), ((), ())),
                           preferred_element_type=jnp.float32)
    anyhit = jnp.max(h, axis=0, keepdims=True)
    out_ref[...] = jnp.where(anyhit > 0, scat, ftb)


_tail_call = pl.pallas_call(
    _tail_body,
    grid=(1,),
    in_specs=[
        pl.BlockSpec((D, TAILBLK), lambda j: (0, TAIL_BASE // TAILBLK)),
        pl.BlockSpec((D, TAILBLK), lambda j: (0, TAIL_BASE // TAILBLK)),
        pl.BlockSpec((B, 1), lambda j: (0, 0)),
        pl.BlockSpec((B, 1), lambda j: (0, 0)),
        pl.BlockSpec((D, B), lambda j: (0, 0)),
    ],
    out_specs=pl.BlockSpec((D, TAILBLK), lambda j: (0, TAIL_BASE // TAILBLK)),
    out_shape=jax.ShapeDtypeStruct((D, N), jnp.float32),
    input_output_aliases={0: 0},
)


def kernel(inputs, inputs_logits, targets, indexes, features):
    del inputs_logits, indexes
    t = targets.astype(jnp.int32)
    xnt, xnst, keepr, keepc = _prep_call(
        inputs.T, t.reshape(B, 1), t.reshape(1, B))
    ft = features.T
    outT = _mm_call(xnst, ft)[0]
    nft = _sc_update(ft, xnt, t, keepr.reshape(B))
    nft = _tail_call(nft, ft, t.reshape(B, 1), keepc, xnt)
    return outT.T, nft.T


# speedup vs baseline: 1.0057x; 1.0057x over previous
import jax
import jax.numpy as jnp
from jax import lax
from jax.experimental import pallas as pl
from jax.experimental.pallas import tpu as pltpu
from jax.experimental.pallas import tpu_sc as plsc

B = 1024
N = 100000
D = 64
TEMP = 0.05
MOM = 0.2

NB = 7168
GRID = (N + NB - 1) // NB

NUM_SUBCORES = 16
LANES = 16
VPR = D // LANES
CHUNK = 512
NFULL = N // CHUNK
TAIL_BASE = NFULL * CHUNK
TAILBLK = 256


def _prep_body(xt_ref, tcol_ref, trow_ref, xnt_ref, xnst_ref,
               keepr_ref, keepc_ref):
    xt = xt_ref[...]
    ss = jnp.sum(xt * xt, axis=0, keepdims=True)
    xnt = xt * lax.rsqrt(jnp.maximum(ss, 1e-24))
    xnt_ref[...] = xnt
    xnst_ref[...] = xnt * (1.0 / TEMP)
    trow = trow_ref[...]
    win = jnp.full((1, B), -1, jnp.int32)
    for c in range(B // 128):
        tcol = tcol_ref[pl.ds(c * 128, 128), :]
        eq = tcol == trow
        iidx = lax.broadcasted_iota(jnp.int32, (128, B), 0) + (c * 128)
        win = jnp.maximum(
            win, jnp.max(jnp.where(eq, iidx, -1), axis=0, keepdims=True))
        jidx = lax.broadcasted_iota(jnp.int32, (128, B), 1)
        winc = jnp.max(jnp.where(eq, jidx, -1), axis=1, keepdims=True)
        ii = lax.broadcasted_iota(jnp.int32, (128, 1), 0) + (c * 128)
        keepc_ref[pl.ds(c * 128, 128), :] = (winc == ii).astype(jnp.int32)
    bidx = lax.broadcasted_iota(jnp.int32, (1, B), 1)
    keepr_ref[...] = (win == bidx).astype(jnp.int32)


_prep_call = pl.pallas_call(
    _prep_body,
    out_shape=[
        jax.ShapeDtypeStruct((D, B), jnp.float32),
        jax.ShapeDtypeStruct((D, B), jnp.float32),
        jax.ShapeDtypeStruct((1, B), jnp.int32),
        jax.ShapeDtypeStruct((B, 1), jnp.int32),
    ],
)


def _mm_body(xnst_ref, ft_ref, out_ref):
    out_ref[...] = lax.dot_general(
        ft_ref[...], xnst_ref[...], (((0,), (0,)), ((), ())),
        preferred_element_type=jnp.float32)


_mm_call = pl.pallas_call(
    _mm_body,
    grid=(GRID,),
    in_specs=[
        pl.BlockSpec((D, B), lambda j: (0, 0)),
        pl.BlockSpec((D, NB), lambda j: (0, j)),
    ],
    out_specs=[
        pl.BlockSpec((NB, B), lambda j: (j, 0)),
    ],
    out_shape=[
        jax.ShapeDtypeStruct((N, B), jnp.float32),
    ],
    compiler_params=pltpu.CompilerParams(
        dimension_semantics=("arbitrary",),
        vmem_limit_bytes=100 * 1024 * 1024,
    ),
)


def _rsqrt_newton(t16):
    i = lax.bitcast_convert_type(t16, jnp.int32)
    y = lax.bitcast_convert_type(jnp.int32(0x5F3759DF) - (i >> 1), jnp.float32)
    for _ in range(3):
        y = y * (1.5 - 0.5 * t16 * y * y)
    return y


def _sc_body(ft_hbm, xnt_hbm, tgt_hbm, keep_hbm, nft_hbm,
             xnt_v, tgt_v, keep_v, chunk_v, hitb_v):
    wid = lax.axis_index("s")
    pltpu.sync_copy(xnt_hbm, xnt_v)
    pltpu.sync_copy(tgt_hbm, tgt_v)
    pltpu.sync_copy(keep_hbm, keep_v)
    row16 = lax.iota(jnp.int32, 16)

    def process_chunk(cbase, width):
        cv = chunk_v.at[:, pl.ds(0, width)]
        pltpu.sync_copy(ft_hbm.at[:, pl.ds(cbase, width)], cv)
        cnt = jnp.int32(0)
        for q in range(B // LANES):
            tg = tgt_v[pl.ds(q * LANES, LANES)]
            kp = keep_v[pl.ds(q * LANES, LANES)]
            m = (tg >= cbase) & (tg < cbase + width) & (kp > 0)
            bv = row16 + (q * LANES)
            plsc.store_compressed(hitb_v.at[pl.ds(cnt, LANES)], bv, mask=m)
            cnt = cnt + plsc.all_reduce_population_count(m)[0]

        def fix_body(i, carry2):
            off = i * LANES
            valid = (row16 + off) < cnt
            b16 = jnp.where(valid, hitb_v[pl.ds(off, LANES)], 0)
            t16 = plsc.load_gather(tgt_v, [b16])
            c16 = t16 - cbase
            for lane in range(LANES):
                @pl.when(off + lane < cnt)
                def _():
                    b_s = b16[lane]
                    c_s = c16[lane]
                    cf = jnp.full((LANES,), c_s, jnp.int32)
                    bf = jnp.full((LANES,), b_s, jnp.int32)
                    us = []
                    ssum = None
                    for q4 in range(VPR):
                        ridx = row16 + (q4 * LANES)
                        g = plsc.load_gather(chunk_v, [ridx, cf])
                        xw = plsc.load_gather(xnt_v, [ridx, bf])
                        u = MOM * g + (1.0 - MOM) * xw
                        us.append(u)
                        p = u * u
                        ssum = p if ssum is None else ssum + p
                    tot = jnp.sum(ssum)
                    y = _rsqrt_newton(jnp.full((LANES,), tot, jnp.float32))
                    for q4 in range(VPR):
                        ridx = row16 + (q4 * LANES)
                        plsc.store_scatter(chunk_v, [ridx, cf], us[q4] * y)
            return carry2

        lax.fori_loop(0, (cnt + LANES - 1) // LANES, fix_body, 0)
        pltpu.sync_copy(cv, nft_hbm.at[:, pl.ds(cbase, width)])

    nmine = jnp.where(wid < (NFULL % NUM_SUBCORES),
                      NFULL // NUM_SUBCORES + 1, NFULL // NUM_SUBCORES)

    def chunk_body(ci, carry):
        cbase = (wid + ci * NUM_SUBCORES) * CHUNK
        process_chunk(cbase, CHUNK)
        return carry

    lax.fori_loop(0, nmine, chunk_body, 0)


_sc_update = pl.kernel(
    _sc_body,
    out_type=jax.ShapeDtypeStruct((D, N), jnp.float32),
    mesh=plsc.VectorSubcoreMesh(
        core_axis_name="c", subcore_axis_name="s",
        num_cores=1, num_subcores=NUM_SUBCORES),
    compiler_params=pltpu.CompilerParams(needs_layout_passes=False),
    scratch_types=[
        pltpu.VMEM((D, B), jnp.float32),
        pltpu.VMEM((B,), jnp.int32),
        pltpu.VMEM((B,), jnp.int32),
        pltpu.VMEM((D, CHUNK), jnp.float32),
        pltpu.VMEM((B + LANES,), jnp.int32),
    ],
)


def _tail_body(nft_ref, ftb_ref, tcol_ref, keepc_ref, xnt_ref, out_ref):
    del nft_ref
    ftb = ftb_ref[...]
    tcol = tcol_ref[...]
    keepc = keepc_ref[...]
    cidx = lax.broadcasted_iota(jnp.int32, (B, TAILBLK), 1) + TAIL_BASE
    h = ((tcol == cidx) & (keepc > 0)).astype(jnp.float32)
    g = lax.dot_general(ftb, h, (((1,), (1,)), ((), ())),
                        preferred_element_type=jnp.float32)
    upd = MOM * g + (1.0 - MOM) * xnt_ref[...]
    ss = jnp.sum(upd * upd, axis=0, keepdims=True)
    upd = upd * lax.rsqrt(jnp.maximum(ss, 1e-24))
    scat = lax.dot_general(upd, h, (((1,), (0,)), ((), ())),
                           preferred_element_type=jnp.float32)
    anyhit = jnp.max(h, axis=0, keepdims=True)
    out_ref[...] = jnp.where(anyhit > 0, scat, ftb)


_tail_call = pl.pallas_call(
    _tail_body,
    grid=(1,),
    in_specs=[
        pl.BlockSpec((D, TAILBLK), lambda j: (0, TAIL_BASE // TAILBLK)),
        pl.BlockSpec((D, TAILBLK), lambda j: (0, TAIL_BASE // TAILBLK)),
        pl.BlockSpec((B, 1), lambda j: (0, 0)),
        pl.BlockSpec((B, 1), lambda j: (0, 0)),
        pl.BlockSpec((D, B), lambda j: (0, 0)),
    ],
    out_specs=pl.BlockSpec((D, TAILBLK), lambda j: (0, TAIL_BASE // TAILBLK)),
    out_shape=jax.ShapeDtypeStruct((D, N), jnp.float32),
    input_output_aliases={0: 0},
)


def kernel(inputs, inputs_logits, targets, indexes, features):
    del inputs_logits, indexes
    t = targets.astype(jnp.int32)
    xnt, xnst, keepr, keepc = _prep_call(
        inputs.T, t.reshape(B, 1), t.reshape(1, B))
    ft = features.T
    outT = _mm_call(xnst, ft)[0]
    nft = _sc_update(ft, xnt, t, keepr.reshape(B))
    nft = _tail_call(nft, ft, t.reshape(B, 1), keepc, xnt)
    return outT.T, nft.T
